# baseline (speedup 1.0000x reference)
import functools

import jax
import jax.numpy as jnp
from jax import lax
from jax.experimental import pallas as pl
from jax.experimental.pallas import tpu as pltpu
from jax.experimental.pallas import tpu_sc as plsc

N_FIELD = 13
VOCAB = 100000
EMB = 16
B = 16384
CONCAT = N_FIELD * EMB
H1, H2 = 64, 32

NC, NS = 2, 16
NW = NC * NS
BPW = B // NW
SUB = 128
NSUB = BPW // SUB
NB2 = B // 128


def _sc_gather_body(utt_hbm, itt_hbm, uidx_hbm, iidx_hbm, xu_hbm, xi_hbm,
                    idx2, rows2, sem_i, sem_g, sem_w):
    wid = lax.axis_index("s") * NC + lax.axis_index("c")
    base = wid * BPW
    for tt, idx_hbm, out_hbm in ((utt_hbm, uidx_hbm, xu_hbm),
                                 (itt_hbm, iidx_hbm, xi_hbm)):
        def idx_src(f, idx_hbm=idx_hbm):
            return idx_hbm.at[pl.ds(f * B + base, BPW)]

        def out_dst(f, out_hbm=out_hbm):
            return out_hbm.at[pl.ds(f * EMB, EMB), pl.ds(wid * NSUB, NSUB), :]

        pltpu.async_copy(idx_src(0), idx2.at[0], sem_i)

        def _field(f, _, tt=tt, idx_src=idx_src, out_dst=out_dst):
            p = f & 1
            pltpu.make_async_copy(idx_src(f), idx2.at[p], sem_i).wait()

            @pl.when(f < N_FIELD - 1)
            def _():
                pltpu.async_copy(idx_src(f + 1), idx2.at[1 - p], sem_i)

            @pl.when(f >= 2)
            def _():
                pltpu.make_async_copy(rows2.at[p], out_dst(f - 2),
                                      sem_w).wait()

            cps = [pltpu.async_copy(
                       tt.at[f, d].at[idx2.at[p].at[pl.ds(k * SUB, SUB)]],
                       rows2.at[p, d, k, :], sem_g)
                   for d in range(EMB) for k in range(NSUB)]
            for c in cps:
                c.wait()
            pltpu.async_copy(rows2.at[p], out_dst(f), sem_w)
            return 0

        lax.fori_loop(0, N_FIELD, _field, 0)
        pltpu.make_async_copy(rows2.at[(N_FIELD - 2) & 1],
                              out_dst(N_FIELD - 2), sem_w).wait()
        pltpu.make_async_copy(rows2.at[(N_FIELD - 1) & 1],
                              out_dst(N_FIELD - 1), sem_w).wait()


_sc_gather = functools.partial(
    pl.kernel,
    out_type=(jax.ShapeDtypeStruct((CONCAT, NB2, 128), jnp.float32),
              jax.ShapeDtypeStruct((CONCAT, NB2, 128), jnp.float32)),
    mesh=plsc.VectorSubcoreMesh(core_axis_name="c", subcore_axis_name="s"),
    scratch_types=[pltpu.VMEM((2, BPW), jnp.int32),
                   pltpu.VMEM((2, EMB, NSUB, 128), jnp.float32),
                   pltpu.SemaphoreType.DMA,
                   pltpu.SemaphoreType.DMA,
                   pltpu.SemaphoreType.DMA],
    compiler_params=pltpu.CompilerParams(use_tc_tiling_on_sc=False),
)(_sc_gather_body)


GRID = 16
BL1 = NB2 // GRID


def _tower_col(x, w1t, b1, w2t, b2):
    h = jnp.maximum(
        jnp.dot(w1t, x, preferred_element_type=jnp.float32) + b1, 0.0)
    return jnp.maximum(
        jnp.dot(w2t, h, preferred_element_type=jnp.float32) + b2, 0.0)


def _tc_dense_body(xu_ref, xi_ref, uw1, ub1, uw2, ub2, iw1, ib1, iw2, ib2,
                   out_ref, acc):
    pdot = jnp.float32(0.0)
    pnu = jnp.float32(0.0)
    pni = jnp.float32(0.0)
    for s in range(BL1):
        hu = _tower_col(xu_ref[:, s, :], uw1[...], ub1[...], uw2[...],
                        ub2[...])
        hi = _tower_col(xi_ref[:, s, :], iw1[...], ib1[...], iw2[...],
                        ib2[...])
        pdot += jnp.sum(hu * hi)
        pnu += jnp.sum(hu * hu)
        pni += jnp.sum(hi * hi)
    i = pl.program_id(0)

    @pl.when(i == 0)
    def _():
        acc[0] = pdot
        acc[1] = pnu
        acc[2] = pni

    @pl.when(i > 0)
    def _():
        acc[0] += pdot
        acc[1] += pnu
        acc[2] += pni

    @pl.when(i == pl.num_programs(0) - 1)
    def _():
        cos = acc[0] / (jnp.sqrt(acc[1]) * jnp.sqrt(acc[2]))
        out_ref[...] = jnp.full((1, 1), jax.nn.sigmoid(cos), jnp.float32)


_tc_dense = pl.pallas_call(
    _tc_dense_body,
    grid=(GRID,),
    in_specs=[
        pl.BlockSpec((CONCAT, BL1, 128), lambda i: (0, i, 0)),
        pl.BlockSpec((CONCAT, BL1, 128), lambda i: (0, i, 0)),
        pl.BlockSpec((H1, CONCAT), lambda i: (0, 0)),
        pl.BlockSpec((H1, 1), lambda i: (0, 0)),
        pl.BlockSpec((H2, H1), lambda i: (0, 0)),
        pl.BlockSpec((H2, 1), lambda i: (0, 0)),
        pl.BlockSpec((H1, CONCAT), lambda i: (0, 0)),
        pl.BlockSpec((H1, 1), lambda i: (0, 0)),
        pl.BlockSpec((H2, H1), lambda i: (0, 0)),
        pl.BlockSpec((H2, 1), lambda i: (0, 0)),
    ],
    out_specs=pl.BlockSpec((1, 1), lambda i: (0, 0)),
    out_shape=jax.ShapeDtypeStruct((1, 1), jnp.float32),
    scratch_shapes=[pltpu.SMEM((3,), jnp.float32)],
)


def kernel(user_indices, item_indices, user_tables, item_tables,
           user_W1, user_b1, user_W2, user_b2,
           item_W1, item_b1, item_W2, item_b2):
    utt = jnp.transpose(user_tables, (0, 2, 1))
    itt = jnp.transpose(item_tables, (0, 2, 1))
    xu, xi = _sc_gather(utt, itt,
                        user_indices.reshape(-1), item_indices.reshape(-1))
    return _tc_dense(xu, xi,
                     user_W1.T, user_b1.reshape(H1, 1),
                     user_W2.T, user_b2.reshape(H2, 1),
                     item_W1.T, item_b1.reshape(H1, 1),
                     item_W2.T, item_b2.reshape(H2, 1))


# speedup vs baseline: 1.1897x; 1.1897x over previous
import functools

import jax
import jax.numpy as jnp
from jax import lax
from jax.experimental import pallas as pl
from jax.experimental.pallas import tpu as pltpu
from jax.experimental.pallas import tpu_sc as plsc

N_FIELD = 13
VOCAB = 100000
EMB = 16
B = 16384
CONCAT = N_FIELD * EMB
H1, H2 = 64, 32

NC, NS = 2, 16
NW = NC * NS
BPW = B // NW
SUB = 128
NSUB = BPW // SUB
NB2 = B // 128


def _sc_gather_body(tt, idx_hbm, out_hbm, idx2, rows2, sem_i, sem_g, sem_w):
    wid = lax.axis_index("s") * NC + lax.axis_index("c")
    base = wid * BPW

    def idx_src(f):
        return idx_hbm.at[pl.ds(f * B + base, BPW)]

    def out_dst(f):
        return out_hbm.at[pl.ds(f * EMB, EMB), pl.ds(wid * NSUB, NSUB), :]

    pltpu.async_copy(idx_src(0), idx2.at[0], sem_i)

    def _field(f, _):
        p = f & 1
        pltpu.make_async_copy(idx_src(f), idx2.at[p], sem_i).wait()

        @pl.when(f < N_FIELD - 1)
        def _():
            pltpu.async_copy(idx_src(f + 1), idx2.at[1 - p], sem_i)

        @pl.when(f >= 2)
        def _():
            pltpu.make_async_copy(rows2.at[p], out_dst(f - 2),
                                  sem_w).wait()

        cps = [pltpu.async_copy(
                   tt.at[f, d].at[idx2.at[p].at[pl.ds(k * SUB, SUB)]],
                   rows2.at[p, d, k, :], sem_g)
               for d in range(EMB) for k in range(NSUB)]
        for c in cps:
            c.wait()
        pltpu.async_copy(rows2.at[p], out_dst(f), sem_w)
        return 0

    lax.fori_loop(0, N_FIELD, _field, 0)
    pltpu.make_async_copy(rows2.at[(N_FIELD - 2) & 1],
                          out_dst(N_FIELD - 2), sem_w).wait()
    pltpu.make_async_copy(rows2.at[(N_FIELD - 1) & 1],
                          out_dst(N_FIELD - 1), sem_w).wait()


_sc_gather = functools.partial(
    pl.kernel,
    out_type=jax.ShapeDtypeStruct((CONCAT, NB2, 128), jnp.float32),
    mesh=plsc.VectorSubcoreMesh(core_axis_name="c", subcore_axis_name="s"),
    scratch_types=[pltpu.VMEM((2, BPW), jnp.int32),
                   pltpu.VMEM((2, EMB, NSUB, 128), jnp.float32),
                   pltpu.SemaphoreType.DMA,
                   pltpu.SemaphoreType.DMA,
                   pltpu.SemaphoreType.DMA],
    compiler_params=pltpu.CompilerParams(use_tc_tiling_on_sc=False),
)(_sc_gather_body)


GRID = 16
BL1 = NB2 // GRID


def _tower_col(x, w1t, b1, w2t, b2):
    h = jnp.maximum(
        jnp.dot(w1t, x, preferred_element_type=jnp.float32) + b1, 0.0)
    return jnp.maximum(
        jnp.dot(w2t, h, preferred_element_type=jnp.float32) + b2, 0.0)


def _tc_dense_body(xu_ref, xi_ref, uw1, ub1, uw2, ub2, iw1, ib1, iw2, ib2,
                   out_ref, acc):
    pdot = jnp.float32(0.0)
    pnu = jnp.float32(0.0)
    pni = jnp.float32(0.0)
    for s in range(BL1):
        hu = _tower_col(xu_ref[:, s, :], uw1[...], ub1[...], uw2[...],
                        ub2[...])
        hi = _tower_col(xi_ref[:, s, :], iw1[...], ib1[...], iw2[...],
                        ib2[...])
        pdot += jnp.sum(hu * hi)
        pnu += jnp.sum(hu * hu)
        pni += jnp.sum(hi * hi)
    i = pl.program_id(0)

    @pl.when(i == 0)
    def _():
        acc[0] = pdot
        acc[1] = pnu
        acc[2] = pni

    @pl.when(i > 0)
    def _():
        acc[0] += pdot
        acc[1] += pnu
        acc[2] += pni

    @pl.when(i == pl.num_programs(0) - 1)
    def _():
        cos = acc[0] / (jnp.sqrt(acc[1]) * jnp.sqrt(acc[2]))
        out_ref[...] = jnp.full((1, 1), jax.nn.sigmoid(cos), jnp.float32)


_tc_dense = pl.pallas_call(
    _tc_dense_body,
    grid=(GRID,),
    in_specs=[
        pl.BlockSpec((CONCAT, BL1, 128), lambda i: (0, i, 0)),
        pl.BlockSpec((CONCAT, BL1, 128), lambda i: (0, i, 0)),
        pl.BlockSpec((H1, CONCAT), lambda i: (0, 0)),
        pl.BlockSpec((H1, 1), lambda i: (0, 0)),
        pl.BlockSpec((H2, H1), lambda i: (0, 0)),
        pl.BlockSpec((H2, 1), lambda i: (0, 0)),
        pl.BlockSpec((H1, CONCAT), lambda i: (0, 0)),
        pl.BlockSpec((H1, 1), lambda i: (0, 0)),
        pl.BlockSpec((H2, H1), lambda i: (0, 0)),
        pl.BlockSpec((H2, 1), lambda i: (0, 0)),
    ],
    out_specs=pl.BlockSpec((1, 1), lambda i: (0, 0)),
    out_shape=jax.ShapeDtypeStruct((1, 1), jnp.float32),
    scratch_shapes=[pltpu.SMEM((3,), jnp.float32)],
)


def kernel(user_indices, item_indices, user_tables, item_tables,
           user_W1, user_b1, user_W2, user_b2,
           item_W1, item_b1, item_W2, item_b2):
    utt = jnp.transpose(user_tables, (0, 2, 1))
    itt = jnp.transpose(item_tables, (0, 2, 1))
    xu = _sc_gather(utt, user_indices.reshape(-1))
    xi = _sc_gather(itt, item_indices.reshape(-1))
    return _tc_dense(xu, xi,
                     user_W1.T, user_b1.reshape(H1, 1),
                     user_W2.T, user_b2.reshape(H2, 1),
                     item_W1.T, item_b1.reshape(H1, 1),
                     item_W2.T, item_b2.reshape(H2, 1))
